# Initial kernel scaffold; baseline (speedup 1.0000x reference)
import functools

import jax
import jax.numpy as jnp
from jax import lax
from jax.experimental import pallas as pl
from jax.experimental.pallas import tpu as pltpu
from jax.experimental.pallas import tpu_sc as plsc

F32 = jnp.float32
BF16 = jnp.bfloat16

B = 4096
NP = 10
HERO_DIM = 128
SKILL_DIM = 64
PH = 1024
PO = 512
TH = 2048
TO = 1024
MH = 2048

NW = 32
HROWS = B * NP
SROWS = B * NP * 4
H_PER_W = HROWS // NW
S_PER_W = SROWS // NW
CH = 128
H_CHUNKS = H_PER_W // CH
S_CHUNKS = S_PER_W // CH


def _leaky(v):
    return jnp.where(v >= 0, v, 0.01 * v)


def _sc_gather(hidx2d, sidx2d, embed_hero, embed_skill):
    mesh = plsc.VectorSubcoreMesh(core_axis_name="c", subcore_axis_name="s")

    @functools.partial(
        pl.kernel,
        out_type=[
            jax.ShapeDtypeStruct((HROWS, HERO_DIM), F32),
            jax.ShapeDtypeStruct((SROWS, SKILL_DIM), F32),
        ],
        mesh=mesh,
        scratch_types=[
            pltpu.VMEM((H_CHUNKS, CH), jnp.int32),
            pltpu.VMEM((S_CHUNKS, CH), jnp.int32),
            pltpu.VMEM((CH, HERO_DIM), F32),
            pltpu.VMEM((CH, SKILL_DIM), F32),
            pltpu.SemaphoreType.DMA,
        ],
    )
    def gather_kernel(hidx_hbm, sidx_hbm, hero_hbm, skill_hbm,
                      hero_out, skill_out, hidx_v, sidx_v, hbuf, sbuf, sem):
        wid = lax.axis_index("s") * 2 + lax.axis_index("c")
        pltpu.sync_copy(hidx_hbm.at[pl.ds(wid * H_CHUNKS, H_CHUNKS)], hidx_v)
        pltpu.sync_copy(sidx_hbm.at[pl.ds(wid * S_CHUNKS, S_CHUNKS)], sidx_v)
        hbase = wid * H_PER_W
        sbase = wid * S_PER_W

        def hero_body(j, carry):
            pltpu.async_copy(hero_hbm.at[hidx_v.at[j]], hbuf, sem).wait()
            pltpu.sync_copy(hbuf, hero_out.at[pl.ds(hbase + j * CH, CH)])
            return carry

        lax.fori_loop(0, H_CHUNKS, hero_body, 0)

        def skill_body(j, carry):
            pltpu.async_copy(skill_hbm.at[sidx_v.at[j]], sbuf, sem).wait()
            pltpu.sync_copy(sbuf, skill_out.at[pl.ds(sbase + j * CH, CH)])
            return carry

        lax.fori_loop(0, S_CHUNKS, skill_body, 0)

    return gather_kernel(hidx2d, sidx2d, embed_hero, embed_skill)


BB = 128
PB = BB * NP


def _mlp_body(h_ref, s_ref, wp1a, wp1b, bp1, wp2, bp2, wt1, bt1, wt2, bt2,
              wm1a, wm1b, bm1, wm2, bm2, o_ref):
    hero = h_ref[0].astype(BF16)
    skill = s_ref[0].astype(BF16)
    a1 = jnp.dot(hero, wp1a[...], preferred_element_type=F32)
    a1 = a1 + jnp.dot(skill, wp1b[...], preferred_element_type=F32)
    p1 = _leaky(a1 + bp1[...]).astype(BF16)
    a2 = jnp.dot(p1, wp2[...], preferred_element_type=F32)
    p = _leaky(a2 + bp2[...]).astype(BF16)
    p3 = p.reshape(BB, NP, PO)
    acc1 = jnp.zeros((BB, TH), F32)
    acc2 = jnp.zeros((BB, TH), F32)
    for i in range(5):
        w = wt1[i * PO:(i + 1) * PO, :]
        acc1 = acc1 + jnp.dot(p3[:, i, :], w, preferred_element_type=F32)
        acc2 = acc2 + jnp.dot(p3[:, 5 + i, :], w, preferred_element_type=F32)
    th1 = _leaky(acc1 + bt1[...]).astype(BF16)
    th2 = _leaky(acc2 + bt1[...]).astype(BF16)
    t1 = _leaky(jnp.dot(th1, wt2[...], preferred_element_type=F32) + bt2[...]).astype(BF16)
    t2 = _leaky(jnp.dot(th2, wt2[...], preferred_element_type=F32) + bt2[...]).astype(BF16)
    mh = _leaky(jnp.dot(t1, wm1a[...], preferred_element_type=F32)
                + jnp.dot(t2, wm1b[...], preferred_element_type=F32)
                + bm1[...]).astype(BF16)
    lg = jnp.dot(mh, wm2[...], preferred_element_type=F32) + bm2[...]
    lg = lg - jnp.max(lg, axis=1, keepdims=True)
    e = jnp.exp(lg)
    o_ref[0] = e / jnp.sum(e, axis=1, keepdims=True)


def _tc_mlp(hero_r, skill_r, wp1a, wp1b, bp1, wp2, bp2, wt1, bt1, wt2, bt2,
            wm1a, wm1b, bm1, wm2, bm2):
    nblk = hero_r.shape[0]

    def full(arr):
        ndim = arr.ndim
        return pl.BlockSpec(arr.shape, lambda i, _n=ndim: (0,) * _n)

    grid_spec = pl.GridSpec(
        grid=(nblk,),
        in_specs=[
            pl.BlockSpec((1, PB, HERO_DIM), lambda i: (i, 0, 0)),
            pl.BlockSpec((1, PB, 4 * SKILL_DIM), lambda i: (i, 0, 0)),
            full(wp1a), full(wp1b), full(bp1), full(wp2), full(bp2),
            full(wt1), full(bt1), full(wt2), full(bt2),
            full(wm1a), full(wm1b), full(bm1), full(wm2), full(bm2),
        ],
        out_specs=pl.BlockSpec((1, BB, 2), lambda i: (i, 0, 0)),
    )
    return pl.pallas_call(
        _mlp_body,
        grid_spec=grid_spec,
        out_shape=jax.ShapeDtypeStruct((nblk, BB, 2), F32),
        compiler_params=pltpu.CompilerParams(
            dimension_semantics=("arbitrary",),
        ),
    )(hero_r, skill_r, wp1a, wp1b, bp1, wp2, bp2, wt1, bt1, wt2, bt2,
      wm1a, wm1b, bm1, wm2, bm2)


def kernel(x, randomize, embed_hero, embed_skill, W_p1, b_p1, W_p2, b_p2,
           W_t1, b_t1, W_t2, b_t2, W_m1, b_m1, W_m2, b_m2):
    del randomize
    hidx2d = x[:, :, 0].reshape(HROWS // CH, CH)
    sidx2d = (x[:, :, 1:] + 1).reshape(SROWS // CH, CH)

    hero_rows, skill_rows = _sc_gather(hidx2d, sidx2d, embed_hero, embed_skill)

    nblk = B // BB
    hero_r = hero_rows.reshape(nblk, PB, HERO_DIM)
    skill_r = skill_rows.reshape(nblk, PB, 4 * SKILL_DIM)

    bf = lambda a: a.astype(BF16)
    out = _tc_mlp(
        hero_r, skill_r,
        bf(W_p1[:HERO_DIM]), bf(W_p1[HERO_DIM:]), b_p1.reshape(1, PH),
        bf(W_p2), b_p2.reshape(1, PO),
        bf(W_t1), b_t1.reshape(1, TH),
        bf(W_t2), b_t2.reshape(1, TO),
        bf(W_m1[:TO]), bf(W_m1[TO:]), b_m1.reshape(1, MH),
        bf(W_m2), b_m2.reshape(1, 2),
    )
    return out.reshape(B, 2)


# speedup vs baseline: 1.8549x; 1.8549x over previous
import functools

import jax
import jax.numpy as jnp
from jax import lax
from jax.experimental import pallas as pl
from jax.experimental.pallas import tpu as pltpu
from jax.experimental.pallas import tpu_sc as plsc

F32 = jnp.float32
BF16 = jnp.bfloat16

B = 4096
NP = 10
HERO_DIM = 128
SKILL_DIM = 64
PH = 1024
PO = 512
TH = 2048
TO = 1024
MH = 2048

NW = 32
HROWS = B * NP
SROWS = B * NP * 4
H_PER_W = HROWS // NW
S_PER_W = SROWS // NW
CH = 128
H_CHUNKS = H_PER_W // CH
S_CHUNKS = S_PER_W // CH


def _leaky(v):
    return jnp.where(v >= 0, v, 0.01 * v)


def _sc_gather(hidx3, sidx3, embed_hero, embed_skill):
    mesh = plsc.VectorSubcoreMesh(core_axis_name="c", subcore_axis_name="s")

    @functools.partial(
        pl.kernel,
        out_type=[
            jax.ShapeDtypeStruct((NW, H_PER_W, HERO_DIM), F32),
            jax.ShapeDtypeStruct((NW, S_PER_W, SKILL_DIM), F32),
        ],
        mesh=mesh,
        scratch_types=[
            pltpu.VMEM((H_CHUNKS, CH), jnp.int32),
            pltpu.VMEM((S_CHUNKS, CH), jnp.int32),
            pltpu.VMEM((CH, HERO_DIM), F32),
            pltpu.VMEM((CH, SKILL_DIM), F32),
            pltpu.SemaphoreType.DMA,
        ],
        compiler_params=pltpu.CompilerParams(use_tc_tiling_on_sc=False),
    )
    def gather_kernel(hidx_hbm, sidx_hbm, hero_hbm, skill_hbm,
                      hero_out, skill_out, hidx_v, sidx_v, hbuf, sbuf, sem):
        wid = lax.axis_index("s") * 2 + lax.axis_index("c")
        pltpu.sync_copy(hidx_hbm.at[wid], hidx_v)
        pltpu.sync_copy(sidx_hbm.at[wid], sidx_v)

        def hero_body(j, carry):
            off = pl.multiple_of(j * CH, CH)
            pltpu.async_copy(hero_hbm.at[hidx_v.at[j]], hbuf, sem).wait()
            pltpu.sync_copy(hbuf, hero_out.at[wid, pl.ds(off, CH)])
            return carry

        lax.fori_loop(0, H_CHUNKS, hero_body, 0)

        def skill_body(j, carry):
            off = pl.multiple_of(j * CH, CH)
            pltpu.async_copy(skill_hbm.at[sidx_v.at[j]], sbuf, sem).wait()
            pltpu.sync_copy(sbuf, skill_out.at[wid, pl.ds(off, CH)])
            return carry

        lax.fori_loop(0, S_CHUNKS, skill_body, 0)

    return gather_kernel(hidx3, sidx3, embed_hero, embed_skill)


BB = 128
PB = BB * NP


def _mlp_body(h_ref, s_ref, wp1a, wp1b, bp1, wp2, bp2, wt1, bt1, wt2, bt2,
              wm1a, wm1b, bm1, wm2, bm2, o_ref):
    hero = h_ref[0].astype(BF16)
    skill = s_ref[0].astype(BF16)
    a1 = jnp.dot(hero, wp1a[...], preferred_element_type=F32)
    a1 = a1 + jnp.dot(skill, wp1b[...], preferred_element_type=F32)
    p1 = _leaky(a1 + bp1[...]).astype(BF16)
    a2 = jnp.dot(p1, wp2[...], preferred_element_type=F32)
    p = _leaky(a2 + bp2[...]).astype(BF16)
    p3 = p.reshape(BB, NP, PO)
    acc1 = jnp.zeros((BB, TH), F32)
    acc2 = jnp.zeros((BB, TH), F32)
    for i in range(5):
        w = wt1[i * PO:(i + 1) * PO, :]
        acc1 = acc1 + jnp.dot(p3[:, i, :], w, preferred_element_type=F32)
        acc2 = acc2 + jnp.dot(p3[:, 5 + i, :], w, preferred_element_type=F32)
    th1 = _leaky(acc1 + bt1[...]).astype(BF16)
    th2 = _leaky(acc2 + bt1[...]).astype(BF16)
    t1 = _leaky(jnp.dot(th1, wt2[...], preferred_element_type=F32) + bt2[...]).astype(BF16)
    t2 = _leaky(jnp.dot(th2, wt2[...], preferred_element_type=F32) + bt2[...]).astype(BF16)
    mh = _leaky(jnp.dot(t1, wm1a[...], preferred_element_type=F32)
                + jnp.dot(t2, wm1b[...], preferred_element_type=F32)
                + bm1[...]).astype(BF16)
    lg = jnp.dot(mh, wm2[...], preferred_element_type=F32) + bm2[...]
    lg = lg - jnp.max(lg, axis=1, keepdims=True)
    e = jnp.exp(lg)
    o_ref[0] = e / jnp.sum(e, axis=1, keepdims=True)


def _tc_mlp(hero_r, skill_r, wp1a, wp1b, bp1, wp2, bp2, wt1, bt1, wt2, bt2,
            wm1a, wm1b, bm1, wm2, bm2):
    nblk = hero_r.shape[0]

    def full(arr):
        ndim = arr.ndim
        return pl.BlockSpec(arr.shape, lambda i, _n=ndim: (0,) * _n)

    grid_spec = pl.GridSpec(
        grid=(nblk,),
        in_specs=[
            pl.BlockSpec((1, PB, HERO_DIM), lambda i: (i, 0, 0)),
            pl.BlockSpec((1, PB, 4 * SKILL_DIM), lambda i: (i, 0, 0)),
            full(wp1a), full(wp1b), full(bp1), full(wp2), full(bp2),
            full(wt1), full(bt1), full(wt2), full(bt2),
            full(wm1a), full(wm1b), full(bm1), full(wm2), full(bm2),
        ],
        out_specs=pl.BlockSpec((1, BB, 2), lambda i: (i, 0, 0)),
    )
    return pl.pallas_call(
        _mlp_body,
        grid_spec=grid_spec,
        out_shape=jax.ShapeDtypeStruct((nblk, BB, 2), F32),
        compiler_params=pltpu.CompilerParams(
            dimension_semantics=("arbitrary",),
        ),
    )(hero_r, skill_r, wp1a, wp1b, bp1, wp2, bp2, wt1, bt1, wt2, bt2,
      wm1a, wm1b, bm1, wm2, bm2)


def kernel(x, randomize, embed_hero, embed_skill, W_p1, b_p1, W_p2, b_p2,
           W_t1, b_t1, W_t2, b_t2, W_m1, b_m1, W_m2, b_m2):
    del randomize
    hidx3 = x[:, :, 0].reshape(NW, H_CHUNKS, CH)
    sidx3 = (x[:, :, 1:] + 1).reshape(NW, S_CHUNKS, CH)

    hero_rows, skill_rows = _sc_gather(hidx3, sidx3, embed_hero, embed_skill)

    nblk = B // BB
    hero_r = hero_rows.reshape(nblk, PB, HERO_DIM)
    skill_r = skill_rows.reshape(nblk, PB, 4 * SKILL_DIM)

    bf = lambda a: a.astype(BF16)
    out = _tc_mlp(
        hero_r, skill_r,
        bf(W_p1[:HERO_DIM]), bf(W_p1[HERO_DIM:]), b_p1.reshape(1, PH),
        bf(W_p2), b_p2.reshape(1, PO),
        bf(W_t1), b_t1.reshape(1, TH),
        bf(W_t2), b_t2.reshape(1, TO),
        bf(W_m1[:TO]), bf(W_m1[TO:]), b_m1.reshape(1, MH),
        bf(W_m2), b_m2.reshape(1, 2),
    )
    return out.reshape(B, 2)


# speedup vs baseline: 1.9690x; 1.0615x over previous
import functools

import jax
import jax.numpy as jnp
from jax import lax
from jax.experimental import pallas as pl
from jax.experimental.pallas import tpu as pltpu
from jax.experimental.pallas import tpu_sc as plsc

F32 = jnp.float32
BF16 = jnp.bfloat16

B = 4096
NP = 10
HERO_DIM = 128
SKILL_DIM = 64
PH = 1024
PO = 512
TH = 2048
TO = 1024
MH = 2048

NW = 32
HROWS = B * NP
SROWS = B * NP * 4
H_PER_W = HROWS // NW
S_PER_W = SROWS // NW
CH = 128
H_CHUNKS = H_PER_W // CH
S_CHUNKS = S_PER_W // CH


def _leaky(v):
    return jnp.where(v >= 0, v, 0.01 * v)


NBUF = 4


def _sc_gather(hidx3, sidx3, embed_hero, embed_skill):
    mesh = plsc.VectorSubcoreMesh(core_axis_name="c", subcore_axis_name="s")

    @functools.partial(
        pl.kernel,
        out_type=[
            jax.ShapeDtypeStruct((NW, H_PER_W, HERO_DIM), BF16),
            jax.ShapeDtypeStruct((NW, S_PER_W, SKILL_DIM), BF16),
        ],
        mesh=mesh,
        scratch_types=[
            pltpu.VMEM((H_CHUNKS, CH), jnp.int32),
            pltpu.VMEM((S_CHUNKS, CH), jnp.int32),
            pltpu.VMEM((NBUF, CH, HERO_DIM), BF16),
            pltpu.VMEM((NBUF, CH, SKILL_DIM), BF16),
            pltpu.SemaphoreType.DMA((NBUF,)),
            pltpu.SemaphoreType.DMA((NBUF,)),
            pltpu.SemaphoreType.DMA((NBUF,)),
            pltpu.SemaphoreType.DMA((NBUF,)),
        ],
        compiler_params=pltpu.CompilerParams(use_tc_tiling_on_sc=False),
    )
    def gather_kernel(hidx_hbm, sidx_hbm, hero_hbm, skill_hbm,
                      hero_out, skill_out, hidx_v, sidx_v, hbuf, sbuf,
                      hg_sem, hs_sem, sg_sem, ss_sem):
        wid = lax.axis_index("s") * 2 + lax.axis_index("c")
        pltpu.sync_copy(hidx_hbm.at[wid], hidx_v)
        pltpu.sync_copy(sidx_hbm.at[wid], sidx_v)

        def run_table(n_chunks, table, idx_v, buf, out, g_sem, s_sem):
            def g_desc(j):
                return pltpu.make_async_copy(
                    table.at[idx_v.at[j]], buf.at[j % NBUF], g_sem.at[j % NBUF])

            def s_desc(j):
                off = pl.multiple_of(j * CH, CH)
                return pltpu.make_async_copy(
                    buf.at[j % NBUF], out.at[wid, pl.ds(off, CH)],
                    s_sem.at[j % NBUF])

            for j in range(NBUF - 1):
                g_desc(j).start()

            def body(j, carry):
                @pl.when(j > 0)
                def _():
                    s_desc(j - 1).wait()

                @pl.when(j + NBUF - 1 < n_chunks)
                def _():
                    g_desc(j + NBUF - 1).start()

                g_desc(j).wait()
                s_desc(j).start()
                return carry

            lax.fori_loop(0, n_chunks, body, 0)
            s_desc(n_chunks - 1).wait()

        run_table(H_CHUNKS, hero_hbm, hidx_v, hbuf, hero_out, hg_sem, hs_sem)
        run_table(S_CHUNKS, skill_hbm, sidx_v, sbuf, skill_out, sg_sem, ss_sem)

    return gather_kernel(hidx3, sidx3, embed_hero, embed_skill)


BB = 128
PB = BB * NP


def _mlp_body(h_ref, s_ref, wp1a, wp1b, bp1, wp2, bp2, wt1, bt1, wt2, bt2,
              wm1a, wm1b, bm1, wm2, bm2, o_ref):
    hero = h_ref[0].astype(BF16)
    skill = s_ref[0].astype(BF16)
    a1 = jnp.dot(hero, wp1a[...], preferred_element_type=F32)
    a1 = a1 + jnp.dot(skill, wp1b[...], preferred_element_type=F32)
    p1 = _leaky(a1 + bp1[...]).astype(BF16)
    a2 = jnp.dot(p1, wp2[...], preferred_element_type=F32)
    p = _leaky(a2 + bp2[...]).astype(BF16)
    p3 = p.reshape(BB, NP, PO)
    acc1 = jnp.zeros((BB, TH), F32)
    acc2 = jnp.zeros((BB, TH), F32)
    for i in range(5):
        w = wt1[i * PO:(i + 1) * PO, :]
        acc1 = acc1 + jnp.dot(p3[:, i, :], w, preferred_element_type=F32)
        acc2 = acc2 + jnp.dot(p3[:, 5 + i, :], w, preferred_element_type=F32)
    th1 = _leaky(acc1 + bt1[...]).astype(BF16)
    th2 = _leaky(acc2 + bt1[...]).astype(BF16)
    t1 = _leaky(jnp.dot(th1, wt2[...], preferred_element_type=F32) + bt2[...]).astype(BF16)
    t2 = _leaky(jnp.dot(th2, wt2[...], preferred_element_type=F32) + bt2[...]).astype(BF16)
    mh = _leaky(jnp.dot(t1, wm1a[...], preferred_element_type=F32)
                + jnp.dot(t2, wm1b[...], preferred_element_type=F32)
                + bm1[...]).astype(BF16)
    lg = jnp.dot(mh, wm2[...], preferred_element_type=F32) + bm2[...]
    lg = lg - jnp.max(lg, axis=1, keepdims=True)
    e = jnp.exp(lg)
    o_ref[0] = e / jnp.sum(e, axis=1, keepdims=True)


def _tc_mlp(hero_r, skill_r, wp1a, wp1b, bp1, wp2, bp2, wt1, bt1, wt2, bt2,
            wm1a, wm1b, bm1, wm2, bm2):
    nblk = hero_r.shape[0]

    def full(arr):
        ndim = arr.ndim
        return pl.BlockSpec(arr.shape, lambda i, _n=ndim: (0,) * _n)

    grid_spec = pl.GridSpec(
        grid=(nblk,),
        in_specs=[
            pl.BlockSpec((1, PB, HERO_DIM), lambda i: (i, 0, 0)),
            pl.BlockSpec((1, PB, 4 * SKILL_DIM), lambda i: (i, 0, 0)),
            full(wp1a), full(wp1b), full(bp1), full(wp2), full(bp2),
            full(wt1), full(bt1), full(wt2), full(bt2),
            full(wm1a), full(wm1b), full(bm1), full(wm2), full(bm2),
        ],
        out_specs=pl.BlockSpec((1, BB, 2), lambda i: (i, 0, 0)),
    )
    return pl.pallas_call(
        _mlp_body,
        grid_spec=grid_spec,
        out_shape=jax.ShapeDtypeStruct((nblk, BB, 2), F32),
        compiler_params=pltpu.CompilerParams(
            dimension_semantics=("arbitrary",),
        ),
    )(hero_r, skill_r, wp1a, wp1b, bp1, wp2, bp2, wt1, bt1, wt2, bt2,
      wm1a, wm1b, bm1, wm2, bm2)


def kernel(x, randomize, embed_hero, embed_skill, W_p1, b_p1, W_p2, b_p2,
           W_t1, b_t1, W_t2, b_t2, W_m1, b_m1, W_m2, b_m2):
    del randomize
    hidx3 = x[:, :, 0].reshape(NW, H_CHUNKS, CH)
    sidx3 = (x[:, :, 1:] + 1).reshape(NW, S_CHUNKS, CH)

    hero_rows, skill_rows = _sc_gather(
        hidx3, sidx3, embed_hero.astype(BF16), embed_skill.astype(BF16))

    nblk = B // BB
    hero_r = hero_rows.reshape(nblk, PB, HERO_DIM)
    skill_r = skill_rows.reshape(nblk, PB, 4 * SKILL_DIM)

    bf = lambda a: a.astype(BF16)
    out = _tc_mlp(
        hero_r, skill_r,
        bf(W_p1[:HERO_DIM]), bf(W_p1[HERO_DIM:]), b_p1.reshape(1, PH),
        bf(W_p2), b_p2.reshape(1, PO),
        bf(W_t1), b_t1.reshape(1, TH),
        bf(W_t2), b_t2.reshape(1, TO),
        bf(W_m1[:TO]), bf(W_m1[TO:]), b_m1.reshape(1, MH),
        bf(W_m2), b_m2.reshape(1, 2),
    )
    return out.reshape(B, 2)


# speedup vs baseline: 2.2357x; 1.1355x over previous
import functools

import jax
import jax.numpy as jnp
from jax import lax
from jax.experimental import pallas as pl
from jax.experimental.pallas import tpu as pltpu
from jax.experimental.pallas import tpu_sc as plsc

F32 = jnp.float32
BF16 = jnp.bfloat16

B = 4096
NP = 10
HERO_DIM = 128
SKILL_DIM = 64
PH = 1024
PO = 512
TH = 2048
TO = 1024
MH = 2048

NW = 32
HROWS = B * NP
SROWS = B * NP * 4
H_PER_W = HROWS // NW
S_PER_W = SROWS // NW
CH = 128
H_CHUNKS = H_PER_W // CH
S_CHUNKS = S_PER_W // CH


def _leaky(v):
    return jnp.where(v >= 0, v, 0.01 * v)


NBUF = 4


def _sc_gather(hidx3, sidx3, embed_hero, embed_skill):
    mesh = plsc.VectorSubcoreMesh(core_axis_name="c", subcore_axis_name="s")

    @functools.partial(
        pl.kernel,
        out_type=[
            jax.ShapeDtypeStruct((NW, H_PER_W, HERO_DIM), BF16),
            jax.ShapeDtypeStruct((NW, S_PER_W, SKILL_DIM), BF16),
        ],
        mesh=mesh,
        scratch_types=[
            pltpu.VMEM((H_CHUNKS, CH), jnp.int32),
            pltpu.VMEM((S_CHUNKS, CH), jnp.int32),
            pltpu.VMEM((NBUF, CH, HERO_DIM), BF16),
            pltpu.VMEM((NBUF, CH, SKILL_DIM), BF16),
            pltpu.SemaphoreType.DMA((NBUF,)),
            pltpu.SemaphoreType.DMA((NBUF,)),
            pltpu.SemaphoreType.DMA((NBUF,)),
            pltpu.SemaphoreType.DMA((NBUF,)),
        ],
        compiler_params=pltpu.CompilerParams(use_tc_tiling_on_sc=False),
    )
    def gather_kernel(hidx_hbm, sidx_hbm, hero_hbm, skill_hbm,
                      hero_out, skill_out, hidx_v, sidx_v, hbuf, sbuf,
                      hg_sem, hs_sem, sg_sem, ss_sem):
        wid = lax.axis_index("s") * 2 + lax.axis_index("c")
        pltpu.sync_copy(hidx_hbm.at[wid], hidx_v)
        pltpu.sync_copy(sidx_hbm.at[wid], sidx_v)

        def run_table(n_chunks, table, idx_v, buf, out, g_sem, s_sem):
            def g_desc(j):
                return pltpu.make_async_copy(
                    table.at[idx_v.at[j]], buf.at[j % NBUF], g_sem.at[j % NBUF])

            def s_desc(j):
                off = pl.multiple_of(j * CH, CH)
                return pltpu.make_async_copy(
                    buf.at[j % NBUF], out.at[wid, pl.ds(off, CH)],
                    s_sem.at[j % NBUF])

            for j in range(NBUF - 1):
                g_desc(j).start()

            def body(j, carry):
                @pl.when(j > 0)
                def _():
                    s_desc(j - 1).wait()

                @pl.when(j + NBUF - 1 < n_chunks)
                def _():
                    g_desc(j + NBUF - 1).start()

                g_desc(j).wait()
                s_desc(j).start()
                return carry

            lax.fori_loop(0, n_chunks, body, 0)
            s_desc(n_chunks - 1).wait()

        run_table(H_CHUNKS, hero_hbm, hidx_v, hbuf, hero_out, hg_sem, hs_sem)
        run_table(S_CHUNKS, skill_hbm, sidx_v, sbuf, skill_out, sg_sem, ss_sem)

    return gather_kernel(hidx3, sidx3, embed_hero, embed_skill)


BB = 128
PB = BB * NP


def _mlp_body(h_ref, s_ref, wp1a, wp1b, bp1, wp2, bp2, wt1, bt1, wt2, bt2,
              wm1a, wm1b, bm1, wm2, bm2, o_ref):
    hero = h_ref[0].astype(BF16)
    skill = s_ref[0].astype(BF16)
    a1 = jnp.dot(hero, wp1a[...], preferred_element_type=F32)
    a1 = a1 + jnp.dot(skill, wp1b[...], preferred_element_type=F32)
    p1 = _leaky(a1 + bp1[...]).astype(BF16)
    a2 = jnp.dot(p1, wp2[...], preferred_element_type=F32)
    p = _leaky(a2 + bp2[...]).astype(BF16)
    acc1 = jnp.zeros((BB, TH), F32)
    acc2 = jnp.zeros((BB, TH), F32)
    for i in range(5):
        w = wt1[i * PO:(i + 1) * PO, :]
        pi = p[i * BB:(i + 1) * BB]
        qi = p[(5 + i) * BB:(6 + i) * BB]
        acc1 = acc1 + jnp.dot(pi, w, preferred_element_type=F32)
        acc2 = acc2 + jnp.dot(qi, w, preferred_element_type=F32)
    th1 = _leaky(acc1 + bt1[...]).astype(BF16)
    th2 = _leaky(acc2 + bt1[...]).astype(BF16)
    t1 = _leaky(jnp.dot(th1, wt2[...], preferred_element_type=F32) + bt2[...]).astype(BF16)
    t2 = _leaky(jnp.dot(th2, wt2[...], preferred_element_type=F32) + bt2[...]).astype(BF16)
    mh = _leaky(jnp.dot(t1, wm1a[...], preferred_element_type=F32)
                + jnp.dot(t2, wm1b[...], preferred_element_type=F32)
                + bm1[...]).astype(BF16)
    lg = jnp.dot(mh, wm2[...], preferred_element_type=F32) + bm2[...]
    lg = lg - jnp.max(lg, axis=1, keepdims=True)
    e = jnp.exp(lg)
    o_ref[0] = e / jnp.sum(e, axis=1, keepdims=True)


def _tc_mlp(hero_r, skill_r, wp1a, wp1b, bp1, wp2, bp2, wt1, bt1, wt2, bt2,
            wm1a, wm1b, bm1, wm2, bm2):
    nblk = hero_r.shape[0]

    def full(arr):
        ndim = arr.ndim
        return pl.BlockSpec(arr.shape, lambda i, _n=ndim: (0,) * _n)

    grid_spec = pl.GridSpec(
        grid=(nblk,),
        in_specs=[
            pl.BlockSpec((1, PB, HERO_DIM), lambda i: (i, 0, 0)),
            pl.BlockSpec((1, PB, 4 * SKILL_DIM), lambda i: (i, 0, 0)),
            full(wp1a), full(wp1b), full(bp1), full(wp2), full(bp2),
            full(wt1), full(bt1), full(wt2), full(bt2),
            full(wm1a), full(wm1b), full(bm1), full(wm2), full(bm2),
        ],
        out_specs=pl.BlockSpec((1, BB, 2), lambda i: (i, 0, 0)),
    )
    return pl.pallas_call(
        _mlp_body,
        grid_spec=grid_spec,
        out_shape=jax.ShapeDtypeStruct((nblk, BB, 2), F32),
        compiler_params=pltpu.CompilerParams(
            dimension_semantics=("arbitrary",),
        ),
    )(hero_r, skill_r, wp1a, wp1b, bp1, wp2, bp2, wt1, bt1, wt2, bt2,
      wm1a, wm1b, bm1, wm2, bm2)


def kernel(x, randomize, embed_hero, embed_skill, W_p1, b_p1, W_p2, b_p2,
           W_t1, b_t1, W_t2, b_t2, W_m1, b_m1, W_m2, b_m2):
    del randomize
    hidx3 = (x[:, :, 0].reshape(NW, BB, NP)
             .transpose(0, 2, 1).reshape(NW, H_CHUNKS, CH))
    sidx3 = ((x[:, :, 1:] + 1).reshape(NW, BB, NP, 4)
             .transpose(0, 2, 1, 3).reshape(NW, S_CHUNKS, CH))

    hero_rows, skill_rows = _sc_gather(
        hidx3, sidx3, embed_hero.astype(BF16), embed_skill.astype(BF16))

    nblk = B // BB
    hero_r = hero_rows.reshape(nblk, PB, HERO_DIM)
    skill_r = skill_rows.reshape(nblk, PB, 4 * SKILL_DIM)

    bf = lambda a: a.astype(BF16)
    out = _tc_mlp(
        hero_r, skill_r,
        bf(W_p1[:HERO_DIM]), bf(W_p1[HERO_DIM:]), b_p1.reshape(1, PH),
        bf(W_p2), b_p2.reshape(1, PO),
        bf(W_t1), b_t1.reshape(1, TH),
        bf(W_t2), b_t2.reshape(1, TO),
        bf(W_m1[:TO]), bf(W_m1[TO:]), b_m1.reshape(1, MH),
        bf(W_m2), b_m2.reshape(1, 2),
    )
    return out.reshape(B, 2)
